# baseline (speedup 1.0000x reference)
import functools

import jax
import jax.numpy as jnp
from jax import lax
from jax.experimental import pallas as pl
from jax.experimental.pallas import tpu as pltpu
from jax.experimental.pallas import tpu_sc as plsc

_NC = 2
_NS = 16
_NW = _NC * _NS
_CHROWS = 128


def _gather_body(tbl_hbm, idx_hbm, out_hbm, idx_v, rows_v, gsem):
    n_ch = idx_hbm.shape[1]
    wid = lax.axis_index("s") * _NC + lax.axis_index("c")
    pltpu.sync_copy(idx_hbm.at[wid], idx_v)
    base = wid * (n_ch * _CHROWS)
    pltpu.make_async_copy(tbl_hbm.at[idx_v.at[0]], rows_v.at[0], gsem).start()

    def body(c, carry):
        @pl.when(c + 1 < n_ch)
        def _():
            pltpu.make_async_copy(
                tbl_hbm.at[idx_v.at[c + 1]], rows_v.at[(c + 1) % 2], gsem
            ).start()

        pltpu.make_async_copy(
            tbl_hbm.at[idx_v.at[c]], rows_v.at[c % 2], gsem
        ).wait()
        pltpu.sync_copy(
            rows_v.at[c % 2], out_hbm.at[pl.ds(base + c * _CHROWS, _CHROWS)]
        )
        return carry

    lax.fori_loop(0, n_ch, body, 0)


def _sc_gather(table, flat_idx):
    n, d = table.shape
    e = flat_idx.shape[0]
    n_ch = e // (_NW * _CHROWS)
    idx3 = flat_idx.reshape(_NW, n_ch, _CHROWS)
    mesh = plsc.VectorSubcoreMesh(
        core_axis_name="c", subcore_axis_name="s", num_cores=_NC, num_subcores=_NS
    )
    run = pl.kernel(
        _gather_body,
        out_type=jax.ShapeDtypeStruct((e, d), table.dtype),
        mesh=mesh,
        scratch_types=[
            pltpu.VMEM((n_ch, _CHROWS), jnp.int32),
            pltpu.VMEM((2, _CHROWS, d), table.dtype),
            pltpu.SemaphoreType.DMA,
        ],
    )
    return run(table, idx3)


def _proj_body(node_ref, w1b_ref, p_ref):
    p_ref[...] = jnp.dot(node_ref[...], w1b_ref[...],
                         preferred_element_type=jnp.float32)


def _tc_proj(node_flat, w1b):
    n, d = node_flat.shape
    return pl.pallas_call(
        _proj_body,
        out_shape=jax.ShapeDtypeStruct((n, d), jnp.float32),
    )(node_flat, w1b)


def _mlp_body(node_ref, g_ref, e_ref, rbf_ref, w1a_ref, w1c_ref,
              w1d_ref, w2_ref, out_ref):
    br = node_ref.shape[1]
    k = e_ref.shape[2]
    d = node_ref.shape[2]
    f32, bf16 = jnp.float32, jnp.bfloat16
    a = jnp.dot(node_ref[0].astype(bf16), w1a_ref[...].astype(bf16),
                preferred_element_type=f32)
    hid = jnp.dot(e_ref[0].reshape(br * k, d).astype(bf16),
                  w1c_ref[...].astype(bf16), preferred_element_type=f32)
    hid += jnp.dot(
        rbf_ref[0].reshape(br * k, rbf_ref.shape[3]).astype(bf16),
        w1d_ref[...].astype(bf16), preferred_element_type=f32,
    )
    hid += g_ref[0].reshape(br * k, d)
    hid = (hid.reshape(br, k, d) + a[:, None, :]).reshape(br * k, d)
    out = jnp.dot(hid.astype(bf16), w2_ref[...].astype(bf16),
                  preferred_element_type=f32)
    out_ref[0] = (out * jax.nn.sigmoid(out)).reshape(br, k, d)


def _tc_mlp(node_embed, gathered, local_edge_embed, rbf_embed, w1a, w1c, w1d, w2):
    b, r, d = node_embed.shape
    k = local_edge_embed.shape[2]
    d_rbf = rbf_embed.shape[3]
    br = 128
    g4 = gathered.reshape(b, r, k, d)
    grid = (b, r // br)
    full = lambda shape: pl.BlockSpec(shape, lambda i, j: (0,) * len(shape))
    return pl.pallas_call(
        _mlp_body,
        grid=grid,
        in_specs=[
            pl.BlockSpec((1, br, d), lambda i, j: (i, j, 0)),
            pl.BlockSpec((1, br, k, d), lambda i, j: (i, j, 0, 0)),
            pl.BlockSpec((1, br, k, d), lambda i, j: (i, j, 0, 0)),
            pl.BlockSpec((1, br, k, d_rbf), lambda i, j: (i, j, 0, 0)),
            full((d, d)),
            full((d, d)),
            full((d_rbf, d)),
            full((d, d)),
        ],
        out_specs=pl.BlockSpec((1, br, k, d), lambda i, j: (i, j, 0, 0)),
        out_shape=jax.ShapeDtypeStruct((b, r, k, d), jnp.float32),
    )(node_embed, g4, local_edge_embed, rbf_embed, w1a, w1c, w1d, w2)


def kernel(node_embed, local_edge_embed, rbf_embed, local_graph, W1, W2):
    b, r, d = node_embed.shape
    w1a, w1b, w1c, w1d = W1[:d], W1[d:2 * d], W1[2 * d:3 * d], W1[3 * d:]
    flat_idx = (jnp.arange(b, dtype=jnp.int32)[:, None, None] * r
                + local_graph.astype(jnp.int32)).reshape(b, -1)
    p = _tc_proj(node_embed.reshape(b * r, d), w1b)
    k = local_edge_embed.shape[2]
    out = jnp.empty((b, r, k, d), jnp.float32)
    for c in range(b):
        g_c = _sc_gather(p, flat_idx[c])
        out_c = _tc_mlp(node_embed[c:c + 1], g_c,
                        local_edge_embed[c:c + 1], rbf_embed[c:c + 1],
                        w1a, w1c, w1d, W2)
        out = lax.dynamic_update_slice(out, out_c, (c, 0, 0, 0))
    return out


# speedup vs baseline: 1.6974x; 1.6974x over previous
import functools

import jax
import jax.numpy as jnp
from jax import lax
from jax.experimental import pallas as pl
from jax.experimental.pallas import tpu as pltpu
from jax.experimental.pallas import tpu_sc as plsc

_NC = 2
_NS = 16
_NW = _NC * _NS
_CHROWS = 128


def _gather_body(tbl_hbm, idx_hbm, out_hbm, idx_v, rows_v, gsem):
    n_ch = idx_hbm.shape[1]
    wid = lax.axis_index("s") * _NC + lax.axis_index("c")
    pltpu.sync_copy(idx_hbm.at[wid], idx_v)
    base = wid * (n_ch * _CHROWS)
    pltpu.make_async_copy(tbl_hbm.at[idx_v.at[0]], rows_v.at[0], gsem).start()

    def body(c, carry):
        @pl.when(c + 1 < n_ch)
        def _():
            pltpu.make_async_copy(
                tbl_hbm.at[idx_v.at[c + 1]], rows_v.at[(c + 1) % 2], gsem
            ).start()

        pltpu.make_async_copy(
            tbl_hbm.at[idx_v.at[c]], rows_v.at[c % 2], gsem
        ).wait()
        pltpu.sync_copy(
            rows_v.at[c % 2], out_hbm.at[pl.ds(base + c * _CHROWS, _CHROWS)]
        )
        return carry

    lax.fori_loop(0, n_ch, body, 0)


def _sc_gather(table, flat_idx):
    n, d = table.shape
    e = flat_idx.shape[0]
    n_ch = e // (_NW * _CHROWS)
    idx3 = flat_idx.reshape(_NW, n_ch, _CHROWS)
    mesh = plsc.VectorSubcoreMesh(
        core_axis_name="c", subcore_axis_name="s", num_cores=_NC, num_subcores=_NS
    )
    run = pl.kernel(
        _gather_body,
        out_type=jax.ShapeDtypeStruct((e, d), table.dtype),
        mesh=mesh,
        scratch_types=[
            pltpu.VMEM((n_ch, _CHROWS), jnp.int32),
            pltpu.VMEM((2, _CHROWS, d), table.dtype),
            pltpu.SemaphoreType.DMA,
        ],
    )
    return run(table, idx3)


def _proj_body(node_ref, w1b_ref, p_ref):
    p_ref[...] = jnp.dot(node_ref[...], w1b_ref[...],
                         preferred_element_type=jnp.float32)


def _tc_proj(node_flat, w1b):
    n, d = node_flat.shape
    return pl.pallas_call(
        _proj_body,
        out_shape=jax.ShapeDtypeStruct((n, d), jnp.float32),
    )(node_flat, w1b)


def _mlp_body(node_ref, g_ref, e_ref, rbf_ref, w1a_ref, w1c_ref,
              w1d_ref, w2_ref, out_ref):
    br = node_ref.shape[1]
    k = e_ref.shape[2]
    d = node_ref.shape[2]
    f32, bf16 = jnp.float32, jnp.bfloat16
    a = jnp.dot(node_ref[0].astype(bf16), w1a_ref[...].astype(bf16),
                preferred_element_type=f32)
    hid = jnp.dot(e_ref[0].reshape(br * k, d).astype(bf16),
                  w1c_ref[...].astype(bf16), preferred_element_type=f32)
    hid += jnp.dot(rbf_ref[0].reshape(br * k, rbf_ref.shape[3]),
                   w1d_ref[...].astype(bf16), preferred_element_type=f32)
    hid += g_ref[0].reshape(br * k, d)
    hid = (hid.reshape(br, k, d) + a[:, None, :]).reshape(br * k, d)
    out = jnp.dot(hid.astype(bf16), w2_ref[...].astype(bf16),
                  preferred_element_type=f32)
    out_ref[0] = (out * jax.nn.sigmoid(out)).reshape(br, k, d)


def _mlp_alias_body(node_ref, g_ref, e_ref, rbf_ref, w1a_ref, w1c_ref,
                    w1d_ref, w2_ref, prev_ref, out_ref):
    del prev_ref
    _mlp_body(node_ref, g_ref, e_ref, rbf_ref, w1a_ref, w1c_ref,
              w1d_ref, w2_ref, out_ref)


def _tc_mlp_half(node_embed, gathered, local_edge_embed, rbf_bf,
                 w1a, w1c, w1d, w2, batch_off, prev_out):
    b, r, d = node_embed.shape
    k = local_edge_embed.shape[2]
    d_rbf = rbf_bf.shape[3]
    br = 128
    g4 = gathered.reshape(b // 2, r, k, d)
    grid = (b // 2, r // br)
    full = lambda shape: pl.BlockSpec(shape, lambda i, j: (0,) * len(shape))
    in_specs = [
        pl.BlockSpec((1, br, d), lambda i, j: (i + batch_off, j, 0)),
        pl.BlockSpec((1, br, k, d), lambda i, j: (i, j, 0, 0)),
        pl.BlockSpec((1, br, k, d), lambda i, j: (i + batch_off, j, 0, 0)),
        pl.BlockSpec((1, br, k, d_rbf), lambda i, j: (i + batch_off, j, 0, 0)),
        full((d, d)),
        full((d, d)),
        full((d_rbf, d)),
        full((d, d)),
    ]
    args = [node_embed, g4, local_edge_embed, rbf_bf, w1a, w1c, w1d, w2]
    out_spec = pl.BlockSpec((1, br, k, d), lambda i, j: (i + batch_off, j, 0, 0))
    out_shape = jax.ShapeDtypeStruct((b, r, k, d), jnp.float32)
    if prev_out is None:
        return pl.pallas_call(
            _mlp_body, grid=grid, in_specs=in_specs,
            out_specs=out_spec, out_shape=out_shape,
        )(*args)
    return pl.pallas_call(
        _mlp_alias_body, grid=grid,
        in_specs=in_specs + [pl.BlockSpec(memory_space=pl.ANY)],
        out_specs=out_spec, out_shape=out_shape,
        input_output_aliases={len(args): 0},
    )(*args, prev_out)


def kernel(node_embed, local_edge_embed, rbf_embed, local_graph, W1, W2):
    b, r, d = node_embed.shape
    k = local_edge_embed.shape[2]
    w1a, w1b, w1c, w1d = W1[:d], W1[d:2 * d], W1[2 * d:3 * d], W1[3 * d:]
    flat_idx = (jnp.arange(b, dtype=jnp.int32)[:, None, None] * r
                + local_graph.astype(jnp.int32)).reshape(2, -1)
    rbf_bf = rbf_embed.astype(jnp.bfloat16)
    p = _tc_proj(node_embed.reshape(b * r, d), w1b)
    g0 = _sc_gather(p, flat_idx[0])
    g1 = _sc_gather(p, flat_idx[1])
    out = _tc_mlp_half(node_embed, g0, local_edge_embed, rbf_bf,
                       w1a, w1c, w1d, W2, 0, None)
    out = _tc_mlp_half(node_embed, g1, local_edge_embed, rbf_bf,
                       w1a, w1c, w1d, W2, b // 2, out)
    return out


# speedup vs baseline: 1.9532x; 1.1507x over previous
import functools

import jax
import jax.numpy as jnp
from jax import lax
from jax.experimental import pallas as pl
from jax.experimental.pallas import tpu as pltpu
from jax.experimental.pallas import tpu_sc as plsc

_NC = 2
_NS = 16
_NW = _NC * _NS
_CHROWS = 128


def _gather_body(tbl_hbm, idx_hbm, out_hbm, idx_v, rows_v, gsem):
    n_ch = idx_hbm.shape[1]
    wid = lax.axis_index("s") * _NC + lax.axis_index("c")
    pltpu.sync_copy(idx_hbm.at[wid], idx_v)
    base = wid * (n_ch * _CHROWS)
    pltpu.make_async_copy(tbl_hbm.at[idx_v.at[0]], rows_v.at[0], gsem).start()

    def body(c, carry):
        @pl.when(c + 1 < n_ch)
        def _():
            pltpu.make_async_copy(
                tbl_hbm.at[idx_v.at[c + 1]], rows_v.at[(c + 1) % 2], gsem
            ).start()

        pltpu.make_async_copy(
            tbl_hbm.at[idx_v.at[c]], rows_v.at[c % 2], gsem
        ).wait()
        pltpu.sync_copy(
            rows_v.at[c % 2], out_hbm.at[pl.ds(base + c * _CHROWS, _CHROWS)]
        )
        return carry

    lax.fori_loop(0, n_ch, body, 0)


def _sc_gather(table, flat_idx):
    n, d = table.shape
    e = flat_idx.shape[0]
    n_ch = e // (_NW * _CHROWS)
    idx3 = flat_idx.reshape(_NW, n_ch, _CHROWS)
    mesh = plsc.VectorSubcoreMesh(
        core_axis_name="c", subcore_axis_name="s", num_cores=_NC, num_subcores=_NS
    )
    run = pl.kernel(
        _gather_body,
        out_type=jax.ShapeDtypeStruct((e, d), table.dtype),
        mesh=mesh,
        scratch_types=[
            pltpu.VMEM((n_ch, _CHROWS), jnp.int32),
            pltpu.VMEM((2, _CHROWS, d), table.dtype),
            pltpu.SemaphoreType.DMA,
        ],
    )
    return run(table, idx3)


def _proj_body(node_ref, w1b_ref, p_ref):
    p_ref[...] = jnp.dot(node_ref[...], w1b_ref[...],
                         preferred_element_type=jnp.float32)


def _tc_proj(node_flat, w1b):
    n, d = node_flat.shape
    return pl.pallas_call(
        _proj_body,
        out_shape=jax.ShapeDtypeStruct((n, d), jnp.float32),
    )(node_flat, w1b)


def _mlp_body(node_ref, g_ref, e_ref, rbf_ref, w1a_ref, w1bc_ref,
              w1d_ref, w2_ref, out_ref):
    br = node_ref.shape[1]
    k = e_ref.shape[2]
    d = node_ref.shape[2]
    f32, bf16 = jnp.float32, jnp.bfloat16
    a = jnp.dot(node_ref[0].astype(bf16), w1a_ref[...].astype(bf16),
                preferred_element_type=f32)
    ge = jnp.concatenate(
        [g_ref[0].reshape(br * k, d).astype(bf16),
         e_ref[0].reshape(br * k, d).astype(bf16)], axis=1)
    hid = jnp.dot(ge, w1bc_ref[...].astype(bf16), preferred_element_type=f32)
    hid += jnp.dot(rbf_ref[0].reshape(br * k, rbf_ref.shape[3]),
                   w1d_ref[...].astype(bf16), preferred_element_type=f32)
    hid = (hid.reshape(br, k, d) + a[:, None, :]).reshape(br * k, d)
    out = jnp.dot(hid.astype(bf16), w2_ref[...].astype(bf16),
                  preferred_element_type=f32)
    out_ref[0] = (out * jax.nn.sigmoid(out)).reshape(br, k, d)


def _tc_mlp(node_embed, gathered, local_edge_embed, rbf_bf, w1a, w1bc, w1d, w2):
    b, r, d = node_embed.shape
    k = local_edge_embed.shape[2]
    d_rbf = rbf_bf.shape[3]
    br = 128
    g4 = gathered.reshape(b, r, k, d)
    grid = (b, r // br)
    full = lambda shape: pl.BlockSpec(shape, lambda i, j: (0,) * len(shape))
    return pl.pallas_call(
        _mlp_body,
        grid=grid,
        in_specs=[
            pl.BlockSpec((1, br, d), lambda i, j: (i, j, 0)),
            pl.BlockSpec((1, br, k, d), lambda i, j: (i, j, 0, 0)),
            pl.BlockSpec((1, br, k, d), lambda i, j: (i, j, 0, 0)),
            pl.BlockSpec((1, br, k, d_rbf), lambda i, j: (i, j, 0, 0)),
            full((d, d)),
            full((2 * d, d)),
            full((d_rbf, d)),
            full((d, d)),
        ],
        out_specs=pl.BlockSpec((1, br, k, d), lambda i, j: (i, j, 0, 0)),
        out_shape=jax.ShapeDtypeStruct((b, r, k, d), jnp.float32),
    )(node_embed, g4, local_edge_embed, rbf_bf, w1a, w1bc, w1d, w2)


def kernel(node_embed, local_edge_embed, rbf_embed, local_graph, W1, W2):
    b, r, d = node_embed.shape
    k = local_edge_embed.shape[2]
    w1a, w1bc, w1d = W1[:d], W1[d:3 * d], W1[3 * d:]
    flat_idx = (jnp.arange(b, dtype=jnp.int32)[:, None, None] * r
                + local_graph.astype(jnp.int32)).reshape(-1)
    rbf_bf = rbf_embed.astype(jnp.bfloat16)
    gathered = _sc_gather(node_embed.reshape(b * r, d), flat_idx)
    return _tc_mlp(node_embed, gathered, local_edge_embed, rbf_bf,
                   w1a, w1bc, w1d, W2)


# speedup vs baseline: 2.1584x; 1.1051x over previous
import functools

import jax
import jax.numpy as jnp
from jax import lax
from jax.experimental import pallas as pl
from jax.experimental.pallas import tpu as pltpu
from jax.experimental.pallas import tpu_sc as plsc

_NC = 2
_NS = 16
_NW = _NC * _NS
_CHROWS = 128


def _gather_body(tbl_hbm, idx_hbm, out_hbm, idx_v, rows_v, tbl_sp, gsem):
    n_ch = idx_hbm.shape[1]
    wid = lax.axis_index("s") * _NC + lax.axis_index("c")

    @pl.when(lax.axis_index("s") == 0)
    def _():
        pltpu.sync_copy(tbl_hbm, tbl_sp)

    pltpu.sync_copy(idx_hbm.at[wid], idx_v)
    plsc.subcore_barrier()
    base = wid * (n_ch * _CHROWS)
    pltpu.make_async_copy(tbl_sp.at[idx_v.at[0]], rows_v.at[0], gsem).start()

    def body(c, carry):
        @pl.when(c + 1 < n_ch)
        def _():
            pltpu.make_async_copy(
                tbl_sp.at[idx_v.at[c + 1]], rows_v.at[(c + 1) % 2], gsem
            ).start()

        pltpu.make_async_copy(
            tbl_sp.at[idx_v.at[c]], rows_v.at[c % 2], gsem
        ).wait()
        pltpu.sync_copy(
            rows_v.at[c % 2], out_hbm.at[pl.ds(base + c * _CHROWS, _CHROWS)]
        )
        return carry

    lax.fori_loop(0, n_ch, body, 0)


def _sc_gather(table, flat_idx):
    n, d = table.shape
    e = flat_idx.shape[0]
    n_ch = e // (_NW * _CHROWS)
    idx3 = flat_idx.reshape(_NW, n_ch, _CHROWS)
    mesh = plsc.VectorSubcoreMesh(
        core_axis_name="c", subcore_axis_name="s", num_cores=_NC, num_subcores=_NS
    )
    run = pl.kernel(
        _gather_body,
        out_type=jax.ShapeDtypeStruct((e, d), table.dtype),
        mesh=mesh,
        scratch_types=[
            pltpu.VMEM((n_ch, _CHROWS), jnp.int32),
            pltpu.VMEM((2, _CHROWS, d), table.dtype),
            pltpu.VMEM_SHARED((n, d), table.dtype),
            pltpu.SemaphoreType.DMA,
        ],
    )
    return run(table, idx3)


def _proj_body(node_ref, w1b_ref, p_ref):
    p_ref[...] = jnp.dot(node_ref[...], w1b_ref[...],
                         preferred_element_type=jnp.float32)


def _tc_proj(node_flat, w1b):
    n, d = node_flat.shape
    return pl.pallas_call(
        _proj_body,
        out_shape=jax.ShapeDtypeStruct((n, d), jnp.float32),
    )(node_flat, w1b)


def _mlp_body(node_ref, g_ref, e_ref, rbf_ref, w1a_ref, w1bc_ref,
              w1d_ref, w2_ref, out_ref):
    br = node_ref.shape[1]
    k = e_ref.shape[2]
    d = node_ref.shape[2]
    f32, bf16 = jnp.float32, jnp.bfloat16
    a = jnp.dot(node_ref[0].astype(bf16), w1a_ref[...].astype(bf16),
                preferred_element_type=f32)
    ge = jnp.concatenate(
        [g_ref[0].reshape(br * k, d).astype(bf16),
         e_ref[0].reshape(br * k, d).astype(bf16)], axis=1)
    hid = jnp.dot(ge, w1bc_ref[...].astype(bf16), preferred_element_type=f32)
    hid += jnp.dot(rbf_ref[0].reshape(br * k, rbf_ref.shape[3]),
                   w1d_ref[...].astype(bf16), preferred_element_type=f32)
    hid = (hid.reshape(br, k, d) + a[:, None, :]).reshape(br * k, d)
    out = jnp.dot(hid.astype(bf16), w2_ref[...].astype(bf16),
                  preferred_element_type=f32)
    out_ref[0] = (out * jax.nn.sigmoid(out)).reshape(br, k, d)


def _tc_mlp(node_embed, gathered, local_edge_embed, rbf_bf, w1a, w1bc, w1d, w2):
    b, r, d = node_embed.shape
    k = local_edge_embed.shape[2]
    d_rbf = rbf_bf.shape[3]
    br = 128
    g4 = gathered.reshape(b, r, k, d)
    grid = (b, r // br)
    full = lambda shape: pl.BlockSpec(shape, lambda i, j: (0,) * len(shape))
    return pl.pallas_call(
        _mlp_body,
        grid=grid,
        in_specs=[
            pl.BlockSpec((1, br, d), lambda i, j: (i, j, 0)),
            pl.BlockSpec((1, br, k, d), lambda i, j: (i, j, 0, 0)),
            pl.BlockSpec((1, br, k, d), lambda i, j: (i, j, 0, 0)),
            pl.BlockSpec((1, br, k, d_rbf), lambda i, j: (i, j, 0, 0)),
            full((d, d)),
            full((2 * d, d)),
            full((d_rbf, d)),
            full((d, d)),
        ],
        out_specs=pl.BlockSpec((1, br, k, d), lambda i, j: (i, j, 0, 0)),
        out_shape=jax.ShapeDtypeStruct((b, r, k, d), jnp.float32),
    )(node_embed, g4, local_edge_embed, rbf_bf, w1a, w1bc, w1d, w2)


def kernel(node_embed, local_edge_embed, rbf_embed, local_graph, W1, W2):
    b, r, d = node_embed.shape
    k = local_edge_embed.shape[2]
    w1a, w1bc, w1d = W1[:d], W1[d:3 * d], W1[3 * d:]
    flat_idx = (jnp.arange(b, dtype=jnp.int32)[:, None, None] * r
                + local_graph.astype(jnp.int32)).reshape(-1)
    rbf_bf = rbf_embed.astype(jnp.bfloat16)
    gathered = _sc_gather(node_embed.reshape(b * r, d), flat_idx)
    return _tc_mlp(node_embed, gathered, local_edge_embed, rbf_bf,
                   w1a, w1bc, w1d, W2)
